# baseline (speedup 1.0000x reference)
import functools

import jax
import jax.numpy as jnp
from jax import lax
from jax.experimental import pallas as pl
from jax.experimental.pallas import tpu as pltpu
from jax.experimental.pallas import tpu_sc as plsc

_NC = 2
_NS = 16
_NW = _NC * _NS
_INV_SCALE = 1.0 / 30.0
_SQRT_HALF = 0.7071067811865476


def _gelu(x):
    return x * (0.5 * (lax.erf(x * _SQRT_HALF) + 1.0))


def _ln(x, g, b):
    m = jnp.mean(x, axis=-1, keepdims=True)
    d = x - m
    v = jnp.mean(d * d, axis=-1, keepdims=True)
    return d * lax.rsqrt(v + 1e-5) * g + b


def _sc_gather(table, idx3):
    nw, nch, ch = idx3.shape
    v, c = table.shape
    e = nw * nch * ch
    per_w = nch * ch

    mesh = plsc.VectorSubcoreMesh(core_axis_name="c", subcore_axis_name="s")

    @functools.partial(
        pl.kernel,
        out_type=jax.ShapeDtypeStruct((e, c), table.dtype),
        mesh=mesh,
        scratch_types=[
            pltpu.VMEM((nch, ch), jnp.int32),
            pltpu.VMEM((ch, c), table.dtype),
            pltpu.VMEM((ch, c), table.dtype),
            pltpu.SemaphoreType.DMA,
            pltpu.SemaphoreType.DMA,
        ],
        compiler_params=pltpu.CompilerParams(use_tc_tiling_on_sc=False),
    )
    def k(table_hbm, idx_hbm, out_hbm, idx_v, buf0, buf1, sem0, sem1):
        wid = lax.axis_index("s") * _NC + lax.axis_index("c")
        base = wid * per_w
        pltpu.sync_copy(idx_hbm.at[wid], idx_v)
        bufs = (buf0, buf1)
        sems = (sem0, sem1)

        def start(chunk, b):
            pltpu.make_async_copy(
                table_hbm.at[idx_v.at[chunk]], bufs[b], sems[b]
            ).start()

        def wait(b):
            pltpu.make_async_copy(
                table_hbm.at[idx_v.at[0]], bufs[b], sems[b]
            ).wait()

        start(0, 0)
        start(1, 1)

        @pl.loop(0, nch // 2)
        def _(p):
            for b in range(2):
                chunk = p * 2 + b
                wait(b)
                pltpu.sync_copy(bufs[b], out_hbm.at[pl.ds(base + chunk * ch, ch)])
                nxt = chunk + 2

                @pl.when(nxt < nch)
                def _():
                    start(nxt, b)

    return k(table, idx3)


def _dot(x, w):
    return jnp.dot(x.astype(jnp.bfloat16), w,
                   preferred_element_type=jnp.float32)


def _body_a(hv_ref, he_ref, g_ref, ma_ref, mv_ref,
            w1a_ref, w1b_ref, w1c_ref, b1_ref, w2_ref, b2_ref, w3_ref, b3_ref,
            l1g_ref, l1b_ref, win_ref, bin_ref, wout_ref, bout_ref,
            l2g_ref, l2b_ref, out_ref, outbf_ref):
    t, cc = hv_ref.shape
    tk = he_ref.shape[0]
    k = tk // t
    hv = hv_ref[...]
    pre = _dot(hv, w1a_ref[...]) + b1_ref[...]
    m = _dot(he_ref[...], w1b_ref[...]) + _dot(g_ref[...], w1c_ref[...])
    x = m.reshape(t, k, cc) + pre[:, None, :]
    x = _gelu(x).reshape(tk, cc)
    x = _gelu(_dot(x, w2_ref[...]) + b2_ref[...])
    x = _dot(x, w3_ref[...]) + b3_ref[...]
    x = x.reshape(t, k, cc) * ma_ref[...][:, :, None]
    dh = jnp.sum(x, axis=1) * _INV_SCALE
    h = _ln(hv + dh, l1g_ref[...], l1b_ref[...])
    f = _gelu(_dot(h, win_ref[...]) + bin_ref[...])
    f = _dot(f, wout_ref[...]) + bout_ref[...]
    y = _ln(h + f, l2g_ref[...], l2b_ref[...]) * mv_ref[...]
    out_ref[...] = y
    outbf_ref[...] = y.astype(jnp.bfloat16)


def _body_b(hv_ref, he_ref, g_ref,
            w1a_ref, w1b_ref, w1c_ref, b1_ref, w2_ref, b2_ref, w3_ref, b3_ref,
            l3g_ref, l3b_ref, out_ref):
    t, cc = hv_ref.shape
    tk = he_ref.shape[0]
    k = tk // t
    he = he_ref[...]
    pre = _dot(hv_ref[...], w1a_ref[...]) + b1_ref[...]
    m = _dot(he, w1b_ref[...]) + _dot(g_ref[...], w1c_ref[...])
    x = m.reshape(t, k, cc) + pre[:, None, :]
    x = _gelu(x).reshape(tk, cc)
    x = _gelu(_dot(x, w2_ref[...]) + b2_ref[...])
    x = _dot(x, w3_ref[...]) + b3_ref[...]
    out_ref[...] = _ln(he + x, l3g_ref[...], l3b_ref[...])


def _tile_spec(t, c):
    return pl.BlockSpec((t, c), lambda i: (i, 0))


def _full_spec(shape):
    return pl.BlockSpec(shape, lambda i: (0, 0))


def kernel(h_V, h_E, E_idx, mask_V, mask_attend,
           W1_w, W1_b, W2_w, W2_b, W3_w, W3_b,
           W11_w, W11_b, W12_w, W12_b, W13_w, W13_b,
           Win_w, Win_b, Wout_w, Wout_b,
           ln1_g, ln1_b, ln2_g, ln2_b, ln3_g, ln3_b):
    bsz, n, k = E_idx.shape
    c = h_V.shape[-1]
    e = n * k
    hv = h_V.reshape(n, c)
    he = h_E.reshape(e, c)
    ma = mask_attend.reshape(n, k)
    mv = mask_V.reshape(n, 1)

    per_w = e // _NW
    ch = 40
    nch = per_w // ch
    idx3 = E_idx.reshape(_NW, nch, ch).astype(jnp.int32)

    def row(x):
        return x.reshape(1, -1)

    def bt(x):
        return x.T.astype(jnp.bfloat16)

    w1a, w1b, w1c = (bt(W1_w[:, :c]), bt(W1_w[:, c:2 * c]), bt(W1_w[:, 2 * c:]))
    w11a, w11b, w11c = (bt(W11_w[:, :c]), bt(W11_w[:, c:2 * c]), bt(W11_w[:, 2 * c:]))
    w2t, w3t, w12t, w13t = bt(W2_w), bt(W3_w), bt(W12_w), bt(W13_w)
    wint, woutt = bt(Win_w), bt(Wout_w)

    t = 400
    grid = (n // t,)
    tk = t * k

    def pack_bf(x):
        m, cc = x.shape
        return lax.bitcast_convert_type(
            x.reshape(m, cc // 2, 2), jnp.int32)

    def unpack_bf(x):
        m, w = x.shape
        return lax.bitcast_convert_type(x, jnp.bfloat16).reshape(m, 2 * w)

    g1 = unpack_bf(_sc_gather(pack_bf(hv.astype(jnp.bfloat16)), idx3))

    hv_new, hv_new_bf = pl.pallas_call(
        _body_a,
        grid=grid,
        in_specs=[
            _tile_spec(t, c),
            _tile_spec(tk, c),
            _tile_spec(tk, c),
            _tile_spec(t, k),
            _tile_spec(t, 1),
            _full_spec((c, c)), _full_spec((c, c)), _full_spec((c, c)),
            _full_spec((1, c)),
            _full_spec((c, c)), _full_spec((1, c)),
            _full_spec((c, c)), _full_spec((1, c)),
            _full_spec((1, c)), _full_spec((1, c)),
            _full_spec((c, 4 * c)), _full_spec((1, 4 * c)),
            _full_spec((4 * c, c)), _full_spec((1, c)),
            _full_spec((1, c)), _full_spec((1, c)),
        ],
        out_specs=[_tile_spec(t, c), _tile_spec(t, c)],
        out_shape=[jax.ShapeDtypeStruct((n, c), jnp.float32),
                   jax.ShapeDtypeStruct((n, c), jnp.bfloat16)],
        compiler_params=pltpu.CompilerParams(
            dimension_semantics=("arbitrary",)),
    )(hv, he, g1, ma, mv,
      w1a, w1b, w1c, row(W1_b), w2t, row(W2_b), w3t, row(W3_b),
      row(ln1_g), row(ln1_b), wint, row(Win_b), woutt, row(Wout_b),
      row(ln2_g), row(ln2_b))

    g2 = unpack_bf(_sc_gather(pack_bf(hv_new_bf), idx3))

    he_out = pl.pallas_call(
        _body_b,
        grid=grid,
        in_specs=[
            _tile_spec(t, c),
            _tile_spec(tk, c),
            _tile_spec(tk, c),
            _full_spec((c, c)), _full_spec((c, c)), _full_spec((c, c)),
            _full_spec((1, c)),
            _full_spec((c, c)), _full_spec((1, c)),
            _full_spec((c, c)), _full_spec((1, c)),
            _full_spec((1, c)), _full_spec((1, c)),
        ],
        out_specs=_tile_spec(tk, c),
        out_shape=jax.ShapeDtypeStruct((e, c), jnp.float32),
        compiler_params=pltpu.CompilerParams(
            dimension_semantics=("arbitrary",)),
    )(hv_new, he, g2,
      w11a, w11b, w11c, row(W11_b), w12t, row(W12_b), w13t, row(W13_b),
      row(ln3_g), row(ln3_b))

    return hv_new.reshape(bsz, n, c), he_out.reshape(bsz, n, k, c)


# speedup vs baseline: 3.2176x; 3.2176x over previous
import functools

import jax
import jax.numpy as jnp
from jax import lax
from jax.experimental import pallas as pl
from jax.experimental.pallas import tpu as pltpu
from jax.experimental.pallas import tpu_sc as plsc

_NC = 2
_NS = 16
_NW = _NC * _NS
_INV_SCALE = 1.0 / 30.0
_SQRT_HALF = 0.7071067811865476


def _gelu(x):
    return x * (0.5 * (lax.erf(x * _SQRT_HALF) + 1.0))


def _ln(x, g, b):
    m = jnp.mean(x, axis=-1, keepdims=True)
    d = x - m
    v = jnp.mean(d * d, axis=-1, keepdims=True)
    return d * lax.rsqrt(v + 1e-5) * g + b


def _dot(x, w):
    return jnp.dot(x, w, preferred_element_type=jnp.float32)


def _sc_gather(table, idx3):
    nw, nch, ch = idx3.shape
    v, c = table.shape
    e = nw * nch * ch
    per_w = nch * ch

    mesh = plsc.VectorSubcoreMesh(core_axis_name="c", subcore_axis_name="s")

    @functools.partial(
        pl.kernel,
        out_type=jax.ShapeDtypeStruct((e, c), table.dtype),
        mesh=mesh,
        scratch_types=[
            pltpu.VMEM((nch, ch), jnp.int32),
            pltpu.VMEM((ch, c), table.dtype),
            pltpu.VMEM((ch, c), table.dtype),
            pltpu.SemaphoreType.DMA,
            pltpu.SemaphoreType.DMA,
        ],
    )
    def k(table_hbm, idx_hbm, out_hbm, idx_v, buf0, buf1, sem0, sem1):
        wid = lax.axis_index("s") * _NC + lax.axis_index("c")
        base = wid * per_w
        pltpu.sync_copy(idx_hbm.at[wid], idx_v)
        bufs = (buf0, buf1)
        sems = (sem0, sem1)

        def start(chunk, b):
            pltpu.make_async_copy(
                table_hbm.at[idx_v.at[chunk]], bufs[b], sems[b]
            ).start()

        def wait(b):
            pltpu.make_async_copy(
                table_hbm.at[idx_v.at[0]], bufs[b], sems[b]
            ).wait()

        start(0, 0)

        @pl.when(nch > 1)
        def _():
            start(1, 1)

        @pl.loop(0, (nch + 1) // 2)
        def _(p):
            for b in range(2):
                chunk = p * 2 + b

                @pl.when(chunk < nch)
                def _():
                    wait(b)
                    pltpu.sync_copy(
                        bufs[b], out_hbm.at[pl.ds(base + chunk * ch, ch)])
                    nxt = chunk + 2

                    @pl.when(nxt < nch)
                    def _():
                        start(nxt, b)

    return k(table, idx3)


def _body_a(acc_ref, hv_ref, he_ref, g_ref, ma_ref, mv_ref,
            w1a_ref, w1b_ref, w1c_ref, b1_ref, w2_ref, b2_ref, w3_ref, b3_ref,
            l1g_ref, l1b_ref, win_ref, bin_ref, wout_ref, bout_ref,
            l2g_ref, l2b_ref, out_ref):
    t, cc = hv_ref.shape
    tk = he_ref.shape[0]
    k = tk // t
    hv = hv_ref[...]
    pre = _dot(hv, w1a_ref[...]) + b1_ref[...]
    m = _dot(he_ref[...], w1b_ref[...]) + _dot(g_ref[...], w1c_ref[...])
    x = m.reshape(t, k, cc) + pre[:, None, :]
    x = _gelu(x).reshape(tk, cc)
    x = _gelu(_dot(x, w2_ref[...]) + b2_ref[...])
    x = _dot(x, w3_ref[...]) + b3_ref[...]
    x = x.reshape(t, k, cc) * ma_ref[...][:, :, None]
    dh = jnp.sum(x, axis=1) * _INV_SCALE
    h = _ln(hv + dh, l1g_ref[...], l1b_ref[...])
    f = _gelu(_dot(h, win_ref[...]) + bin_ref[...])
    f = _dot(f, wout_ref[...]) + bout_ref[...]
    out_ref[...] = _ln(h + f, l2g_ref[...], l2b_ref[...]) * mv_ref[...]


def _body_b(acc_ref, hv_ref, he_ref, g_ref,
            w1a_ref, w1b_ref, w1c_ref, b1_ref, w2_ref, b2_ref, w3_ref, b3_ref,
            l3g_ref, l3b_ref, out_ref):
    t, cc = hv_ref.shape
    tk = he_ref.shape[0]
    k = tk // t
    he = he_ref[...]
    pre = _dot(hv_ref[...], w1a_ref[...]) + b1_ref[...]
    m = _dot(he, w1b_ref[...]) + _dot(g_ref[...], w1c_ref[...])
    x = m.reshape(t, k, cc) + pre[:, None, :]
    x = _gelu(x).reshape(tk, cc)
    x = _gelu(_dot(x, w2_ref[...]) + b2_ref[...])
    x = _dot(x, w3_ref[...]) + b3_ref[...]
    out_ref[...] = _ln(he + x, l3g_ref[...], l3b_ref[...])


def kernel(h_V, h_E, E_idx, mask_V, mask_attend,
           W1_w, W1_b, W2_w, W2_b, W3_w, W3_b,
           W11_w, W11_b, W12_w, W12_b, W13_w, W13_b,
           Win_w, Win_b, Wout_w, Wout_b,
           ln1_g, ln1_b, ln2_g, ln2_b, ln3_g, ln3_b):
    bsz, n, k = E_idx.shape
    c = h_V.shape[-1]
    e = n * k
    hv = h_V.reshape(n, c)
    he = h_E.reshape(e, c)
    ma = mask_attend.reshape(n, k)
    mv = mask_V.reshape(n, 1)
    idx = E_idx.reshape(e).astype(jnp.int32)

    ns = 10
    t = 200
    s_nodes = n // ns
    tps = s_nodes // t
    s_edges = s_nodes * k
    tk = t * k
    ch = 40
    per_w = s_edges // _NW
    nch = per_w // ch

    idx3 = [idx[s * s_edges:(s + 1) * s_edges].reshape(_NW, nch, ch)
            for s in range(ns)]

    def row(x):
        return x.reshape(1, -1)

    w1a, w1b, w1c = (W1_w[:, :c].T, W1_w[:, c:2 * c].T, W1_w[:, 2 * c:].T)
    w11a, w11b, w11c = (W11_w[:, :c].T, W11_w[:, c:2 * c].T, W11_w[:, 2 * c:].T)
    w2t, w3t, w12t, w13t = W2_w.T, W3_w.T, W12_w.T, W13_w.T
    wint, woutt = Win_w.T, Wout_w.T

    wa = (w1a, w1b, w1c, row(W1_b), w2t, row(W2_b), w3t, row(W3_b),
          row(ln1_g), row(ln1_b), wint, row(Win_b), woutt, row(Wout_b),
          row(ln2_g), row(ln2_b))
    wb = (w11a, w11b, w11c, row(W11_b), w12t, row(W12_b), w13t, row(W13_b),
          row(ln3_g), row(ln3_b))

    def full(x):
        return pl.BlockSpec(x.shape, lambda i: tuple(0 for _ in x.shape))

    any_spec = pl.BlockSpec(memory_space=pl.ANY)

    def node_spec(s, rows):
        return pl.BlockSpec((rows, c), lambda i, s=s: (s * tps + i, 0))

    def phase_a(s, acc, g1_s):
        specs = [any_spec,
                 node_spec(s, t),
                 pl.BlockSpec((tk, c), lambda i, s=s: (s * tps + i, 0)),
                 pl.BlockSpec((tk, c), lambda i: (i, 0)),
                 pl.BlockSpec((t, k), lambda i, s=s: (s * tps + i, 0)),
                 pl.BlockSpec((t, 1), lambda i, s=s: (s * tps + i, 0)),
                 ] + [full(w) for w in wa]
        return pl.pallas_call(
            _body_a,
            grid=(tps,),
            in_specs=specs,
            out_specs=node_spec(s, t),
            out_shape=jax.ShapeDtypeStruct((n, c), jnp.float32),
            input_output_aliases={0: 0},
            compiler_params=pltpu.CompilerParams(
                dimension_semantics=("arbitrary",)),
        )(acc, hv, he, g1_s, ma, mv, *wa)

    def phase_b(s, acc, hv_new, g2_s):
        specs = [any_spec,
                 node_spec(s, t),
                 pl.BlockSpec((tk, c), lambda i, s=s: (s * tps + i, 0)),
                 pl.BlockSpec((tk, c), lambda i: (i, 0)),
                 ] + [full(w) for w in wb]
        return pl.pallas_call(
            _body_b,
            grid=(tps,),
            in_specs=specs,
            out_specs=pl.BlockSpec((tk, c), lambda i, s=s: (s * tps + i, 0)),
            out_shape=jax.ShapeDtypeStruct((e, c), jnp.float32),
            input_output_aliases={0: 0},
            compiler_params=pltpu.CompilerParams(
                dimension_semantics=("arbitrary",)),
        )(acc, hv_new, he, g2_s, *wb)

    g1 = [_sc_gather(hv, idx3[s]) for s in range(ns)]

    acc = jnp.zeros((n, c), jnp.float32)
    for s in range(ns):
        acc = phase_a(s, acc, g1[s])
    hv_new = acc

    g2 = [_sc_gather(hv_new, idx3[s]) for s in range(ns)]

    acc_e = jnp.zeros((e, c), jnp.float32)
    for s in range(ns):
        acc_e = phase_b(s, acc_e, hv_new, g2[s])

    return hv_new.reshape(bsz, n, c), acc_e.reshape(bsz, n, k, c)


# speedup vs baseline: 3.4175x; 1.0621x over previous
import functools

import jax
import jax.numpy as jnp
from jax import lax
from jax.experimental import pallas as pl
from jax.experimental.pallas import tpu as pltpu
from jax.experimental.pallas import tpu_sc as plsc

_NC = 2
_NS = 16
_NW = _NC * _NS
_INV_SCALE = 1.0 / 30.0
_SQRT_HALF = 0.7071067811865476


def _gelu(x):
    return x * (0.5 * (lax.erf(x * _SQRT_HALF) + 1.0))


def _ln(x, g, b):
    m = jnp.mean(x, axis=-1, keepdims=True)
    d = x - m
    v = jnp.mean(d * d, axis=-1, keepdims=True)
    return d * lax.rsqrt(v + 1e-5) * g + b


def _dot(x, w):
    return jnp.dot(x, w, preferred_element_type=jnp.float32)


def _sc_gather(table, idx3):
    nw, nch, ch = idx3.shape
    v, c = table.shape
    e = nw * nch * ch
    per_w = nch * ch

    mesh = plsc.VectorSubcoreMesh(core_axis_name="c", subcore_axis_name="s")

    @functools.partial(
        pl.kernel,
        out_type=jax.ShapeDtypeStruct((e, c), table.dtype),
        mesh=mesh,
        scratch_types=[
            pltpu.VMEM((nch, ch), jnp.int32),
            pltpu.VMEM((ch, c), table.dtype),
            pltpu.VMEM((ch, c), table.dtype),
            pltpu.SemaphoreType.DMA,
            pltpu.SemaphoreType.DMA,
        ],
    )
    def k(table_hbm, idx_hbm, out_hbm, idx_v, buf0, buf1, sem0, sem1):
        wid = lax.axis_index("s") * _NC + lax.axis_index("c")
        base = wid * per_w
        pltpu.sync_copy(idx_hbm.at[wid], idx_v)
        bufs = (buf0, buf1)
        sems = (sem0, sem1)

        def start(chunk, b):
            pltpu.make_async_copy(
                table_hbm.at[idx_v.at[chunk]], bufs[b], sems[b]
            ).start()

        def wait(b):
            pltpu.make_async_copy(
                table_hbm.at[idx_v.at[0]], bufs[b], sems[b]
            ).wait()

        start(0, 0)

        @pl.when(nch > 1)
        def _():
            start(1, 1)

        @pl.loop(0, (nch + 1) // 2)
        def _(p):
            for b in range(2):
                chunk = p * 2 + b

                @pl.when(chunk < nch)
                def _():
                    wait(b)
                    pltpu.sync_copy(
                        bufs[b], out_hbm.at[pl.ds(base + chunk * ch, ch)])
                    nxt = chunk + 2

                    @pl.when(nxt < nch)
                    def _():
                        start(nxt, b)

    return k(table, idx3)


def _body_a(acc_ref, hv_ref, he_ref, g_ref, ma_ref, mv_ref,
            w1a_ref, w1b_ref, w1c_ref, b1_ref, w2_ref, b2_ref, w3_ref, b3_ref,
            l1g_ref, l1b_ref, win_ref, bin_ref, wout_ref, bout_ref,
            l2g_ref, l2b_ref, out_ref):
    t, cc = hv_ref.shape
    tk = he_ref.shape[0]
    k = tk // t
    hv = hv_ref[...]
    pre = _dot(hv, w1a_ref[...]) + b1_ref[...]
    m = _dot(he_ref[...], w1b_ref[...]) + _dot(g_ref[...], w1c_ref[...])
    x = m.reshape(t, k, cc) + pre[:, None, :]
    x = _gelu(x).reshape(tk, cc)
    x = _gelu(_dot(x, w2_ref[...]) + b2_ref[...])
    x = _dot(x, w3_ref[...]) + b3_ref[...]
    x = x.reshape(t, k, cc) * ma_ref[...][:, :, None]
    dh = jnp.sum(x, axis=1) * _INV_SCALE
    h = _ln(hv + dh, l1g_ref[...], l1b_ref[...])
    f = _gelu(_dot(h, win_ref[...]) + bin_ref[...])
    f = _dot(f, wout_ref[...]) + bout_ref[...]
    out_ref[...] = _ln(h + f, l2g_ref[...], l2b_ref[...]) * mv_ref[...]


def _body_b(acc_ref, hv_ref, he_ref, g_ref,
            w1a_ref, w1b_ref, w1c_ref, b1_ref, w2_ref, b2_ref, w3_ref, b3_ref,
            l3g_ref, l3b_ref, out_ref):
    t, cc = hv_ref.shape
    tk = he_ref.shape[0]
    k = tk // t
    he = he_ref[...]
    pre = _dot(hv_ref[...], w1a_ref[...]) + b1_ref[...]
    m = _dot(he, w1b_ref[...]) + _dot(g_ref[...], w1c_ref[...])
    x = m.reshape(t, k, cc) + pre[:, None, :]
    x = _gelu(x).reshape(tk, cc)
    x = _gelu(_dot(x, w2_ref[...]) + b2_ref[...])
    x = _dot(x, w3_ref[...]) + b3_ref[...]
    out_ref[...] = _ln(he + x, l3g_ref[...], l3b_ref[...])


def kernel(h_V, h_E, E_idx, mask_V, mask_attend,
           W1_w, W1_b, W2_w, W2_b, W3_w, W3_b,
           W11_w, W11_b, W12_w, W12_b, W13_w, W13_b,
           Win_w, Win_b, Wout_w, Wout_b,
           ln1_g, ln1_b, ln2_g, ln2_b, ln3_g, ln3_b):
    bsz, n, k = E_idx.shape
    c = h_V.shape[-1]
    e = n * k
    hv = h_V.reshape(n, c)
    he = h_E.reshape(e, c)
    ma = mask_attend.reshape(n, k)
    mv = mask_V.reshape(n, 1)
    idx = E_idx.reshape(e).astype(jnp.int32)

    ns = 10
    t = 200
    s_nodes = n // ns
    tps = s_nodes // t
    s_edges = s_nodes * k
    tk = t * k
    ch = 40
    per_w = s_edges // _NW
    nch = per_w // ch

    sc_sizes = [1, 1, 2, 3, 3]
    sc_starts = []
    slice_call = {}
    s0 = 0
    for j, m in enumerate(sc_sizes):
        sc_starts.append(s0)
        for u in range(m):
            slice_call[s0 + u] = (j, u)
        s0 += m

    idx3 = [idx[sc_starts[j] * s_edges:(sc_starts[j] + m) * s_edges]
            .reshape(_NW, m * nch, ch)
            for j, m in enumerate(sc_sizes)]

    def row(x):
        return x.reshape(1, -1)

    w1a, w1b, w1c = (W1_w[:, :c].T, W1_w[:, c:2 * c].T, W1_w[:, 2 * c:].T)
    w11a, w11b, w11c = (W11_w[:, :c].T, W11_w[:, c:2 * c].T, W11_w[:, 2 * c:].T)
    w2t, w3t, w12t, w13t = W2_w.T, W3_w.T, W12_w.T, W13_w.T
    wint, woutt = Win_w.T, Wout_w.T

    wa = (w1a, w1b, w1c, row(W1_b), w2t, row(W2_b), w3t, row(W3_b),
          row(ln1_g), row(ln1_b), wint, row(Win_b), woutt, row(Wout_b),
          row(ln2_g), row(ln2_b))
    wb = (w11a, w11b, w11c, row(W11_b), w12t, row(W12_b), w13t, row(W13_b),
          row(ln3_g), row(ln3_b))

    def full(x):
        return pl.BlockSpec(x.shape, lambda i: tuple(0 for _ in x.shape))

    any_spec = pl.BlockSpec(memory_space=pl.ANY)

    def node_spec(s, rows):
        return pl.BlockSpec((rows, c), lambda i, s=s: (s * tps + i, 0))

    def phase_a(s, acc, g1_s, local):
        specs = [node_spec(s, t),
                 pl.BlockSpec((tk, c), lambda i, s=s: (s * tps + i, 0)),
                 pl.BlockSpec((tk, c), lambda i, u=local: (u * tps + i, 0)),
                 pl.BlockSpec((t, k), lambda i, s=s: (s * tps + i, 0)),
                 pl.BlockSpec((t, 1), lambda i, s=s: (s * tps + i, 0)),
                 ] + [full(w) for w in wa]
        args = [hv, he, g1_s, ma, mv] + list(wa)
        alias = {}
        if acc is not None:
            specs = [any_spec] + specs
            args = [acc] + args
            alias = {0: 0}
        body = _body_a if acc is not None else (
            lambda *rs: _body_a(None, *rs))
        return pl.pallas_call(
            body,
            grid=(tps,),
            in_specs=specs,
            out_specs=node_spec(s, t),
            out_shape=jax.ShapeDtypeStruct((n, c), jnp.float32),
            input_output_aliases=alias,
            compiler_params=pltpu.CompilerParams(
                dimension_semantics=("arbitrary",)),
        )(*args)

    def phase_b(s, acc, hv_new, g2_s, local):
        specs = [node_spec(s, t),
                 pl.BlockSpec((tk, c), lambda i, s=s: (s * tps + i, 0)),
                 pl.BlockSpec((tk, c), lambda i, u=local: (u * tps + i, 0)),
                 ] + [full(w) for w in wb]
        args = [hv_new, he, g2_s] + list(wb)
        alias = {}
        if acc is not None:
            specs = [any_spec] + specs
            args = [acc] + args
            alias = {0: 0}
        body = _body_b if acc is not None else (
            lambda *rs: _body_b(None, *rs))
        return pl.pallas_call(
            body,
            grid=(tps,),
            in_specs=specs,
            out_specs=pl.BlockSpec((tk, c), lambda i, s=s: (s * tps + i, 0)),
            out_shape=jax.ShapeDtypeStruct((e, c), jnp.float32),
            input_output_aliases=alias,
            compiler_params=pltpu.CompilerParams(
                dimension_semantics=("arbitrary",)),
        )(*args)

    g1 = [_sc_gather(hv, idx3[j]) for j in range(len(sc_sizes))]

    acc = None
    for s in range(ns):
        j, local = slice_call[s]
        acc = phase_a(s, acc, g1[j], local)
    hv_new = acc

    g2 = [_sc_gather(hv_new, idx3[j]) for j in range(len(sc_sizes))]

    acc_e = None
    for s in range(ns):
        j, local = slice_call[s]
        acc_e = phase_b(s, acc_e, hv_new, g2[j], local)

    return hv_new.reshape(bsz, n, c), acc_e.reshape(bsz, n, k, c)
